# baseline (speedup 1.0000x reference)
import functools

import jax
import jax.numpy as jnp
from jax import lax
from jax.experimental import pallas as pl
from jax.experimental.pallas import tpu as pltpu
from jax.experimental.pallas import tpu_sc as plsc

N = 10000
E = 320000
D = 128
HD = D // 2

NC = 2
NS = 16
NW = NC * NS

CH = 80
CHS = 128
NSLOT = 4

NPAD = 10240
DEG_PT = NPAD // NS
ROWS_PT = NPAD // NS
ZROWS = 64

EPT_DEG = E // NW
NCH_DEG = EPT_DEG // CH
NBODY_DEG = NCH_DEG // NSLOT
NTAIL_DEG = NCH_DEG - NBODY_DEG * NSLOT

EPT_SC = E // NS
NCH_SC = EPT_SC // CHS
NBODY_SC = NCH_SC // NSLOT
NTAIL_SC = NCH_SC - NBODY_SC * NSLOT
CHT = EPT_SC - NCH_SC * CHS

_mesh = plsc.VectorSubcoreMesh(core_axis_name="c", subcore_axis_name="s")


@functools.partial(
    pl.kernel,
    out_type=jax.ShapeDtypeStruct((NC, NPAD), jnp.float32),
    mesh=_mesh,
    scratch_types=(
        [pltpu.VMEM((CH,), jnp.int32)] * (2 * NSLOT)
        + [
            pltpu.VMEM((CH,), jnp.float32),
            pltpu.VMEM((DEG_PT,), jnp.float32),
            pltpu.VMEM_SHARED((NPAD,), jnp.float32),
        ]
        + [pltpu.SemaphoreType.DMA] * (NSLOT + 1)
    ),
)
def _deg_sc(row_hbm, col_hbm, out_hbm, *refs):
    ridx = refs[0:NSLOT]
    cidx = refs[NSLOT:2 * NSLOT]
    ones_v, zeros_v, deg_sh = refs[2 * NSLOT:2 * NSLOT + 3]
    semi = refs[2 * NSLOT + 3:3 * NSLOT + 3]
    sems = refs[3 * NSLOT + 3]

    c = lax.axis_index("c")
    s = lax.axis_index("s")
    wid = s * NC + c

    def fill(i, _):
        ones_v[pl.ds(i * 16, 16)] = jnp.ones((16,), jnp.float32)
        return 0

    lax.fori_loop(0, CH // 16, fill, 0)

    def zfill(i, _):
        zeros_v[pl.ds(i * 16, 16)] = jnp.zeros((16,), jnp.float32)
        return 0

    lax.fori_loop(0, DEG_PT // 16, zfill, 0)
    pltpu.sync_copy(zeros_v, deg_sh.at[pl.ds(s * DEG_PT, DEG_PT)])
    plsc.subcore_barrier()

    def chunk_base(i):
        return wid * EPT_DEG + i * CH

    def body(j, _):
        di = []
        for b in range(NSLOT):
            base = chunk_base(j * NSLOT + b)
            di.append(pltpu.async_copy(row_hbm.at[pl.ds(base, CH)],
                                       ridx[b], semi[b]))
            di.append(pltpu.async_copy(col_hbm.at[pl.ds(base, CH)],
                                       cidx[b], semi[b]))
        sc = []
        for b in range(NSLOT):
            di[2 * b].wait()
            di[2 * b + 1].wait()
            sc.append(pltpu.async_copy(ones_v, deg_sh.at[ridx[b]], sems,
                                       add=True))
            sc.append(pltpu.async_copy(ones_v, deg_sh.at[cidx[b]], sems,
                                       add=True))
        for d in sc:
            d.wait()
        return 0

    lax.fori_loop(0, NBODY_DEG, body, 0)

    for t in range(NTAIL_DEG):
        base = chunk_base(NBODY_DEG * NSLOT + t)
        pltpu.sync_copy(row_hbm.at[pl.ds(base, CH)], ridx[0])
        pltpu.sync_copy(col_hbm.at[pl.ds(base, CH)], cidx[0])
        pltpu.sync_copy(ones_v, deg_sh.at[ridx[0]], add=True)
        pltpu.sync_copy(ones_v, deg_sh.at[cidx[0]], add=True)

    plsc.subcore_barrier()
    pltpu.sync_copy(deg_sh.at[pl.ds(s * DEG_PT, DEG_PT)],
                    out_hbm.at[c, pl.ds(s * DEG_PT, DEG_PT)])


@functools.partial(
    pl.kernel,
    out_type=jax.ShapeDtypeStruct((NPAD, NC, HD), jnp.float32),
    mesh=_mesh,
    compiler_params=pltpu.CompilerParams(use_tc_tiling_on_sc=False),
    scratch_types=(
        [pltpu.VMEM((CHS,), jnp.int32)] * (6 * NSLOT)
        + [pltpu.VMEM((CHS, HD), jnp.float32)] * (2 * NSLOT)
        + [pltpu.VMEM((CHT,), jnp.int32)] * 4
        + [pltpu.VMEM((CHT, HD), jnp.float32)] * 2
        + [
            pltpu.VMEM((ZROWS, HD), jnp.float32),
            pltpu.VMEM_SHARED((NPAD, HD), jnp.float32),
        ]
        + [pltpu.SemaphoreType.DMA] * (3 * NSLOT + 1)
    ),
)
def _scat_sc(row_hbm, col_hbm, gflat_hbm, out_hbm, *refs):
    ridx = refs[0:NSLOT]
    cidx = refs[NSLOT:2 * NSLOT]
    radj = refs[2 * NSLOT:3 * NSLOT]
    cadj = refs[3 * NSLOT:4 * NSLOT]
    sridx = refs[4 * NSLOT:5 * NSLOT]
    scidx = refs[5 * NSLOT:6 * NSLOT]
    bufa = refs[6 * NSLOT:7 * NSLOT]
    bufb = refs[7 * NSLOT:8 * NSLOT]
    tidx = refs[8 * NSLOT:8 * NSLOT + 4]
    tbuf = refs[8 * NSLOT + 4:8 * NSLOT + 6]
    zbuf, acc_sh = refs[8 * NSLOT + 6:8 * NSLOT + 8]
    semi = refs[8 * NSLOT + 8:9 * NSLOT + 8]
    semg = refs[9 * NSLOT + 8:10 * NSLOT + 8]
    semh = refs[10 * NSLOT + 8:11 * NSLOT + 8]
    sems = refs[11 * NSLOT + 8]

    c = lax.axis_index("c")
    s = lax.axis_index("s")

    def zfill(i, _):
        j = i // (HD // 16)
        k = i % (HD // 16)
        zbuf[j, pl.ds(k * 16, 16)] = jnp.zeros((16,), jnp.float32)
        return 0

    lax.fori_loop(0, ZROWS * (HD // 16), zfill, 0)

    def zinit(t, _):
        pltpu.sync_copy(zbuf, acc_sh.at[pl.ds(s * ROWS_PT + t * ZROWS, ZROWS)])
        return 0

    lax.fori_loop(0, ROWS_PT // ZROWS, zinit, 0)
    plsc.subcore_barrier()

    def chunk_base(i):
        return s * EPT_SC + i * CHS

    def adjust(b):
        def adj(k, _):
            sl = pl.ds(k * 16, 16)
            radj[b][sl] = ridx[b][sl] * 2 + c
            cadj[b][sl] = cidx[b][sl] * 2 + c
            return 0

        lax.fori_loop(0, CHS // 16, adj, 0)

    def drain_scatters(b):
        pltpu.make_async_copy(bufa[b], acc_sh.at[scidx[b]], sems).wait()
        pltpu.make_async_copy(bufb[b], acc_sh.at[sridx[b]], sems).wait()

    def body(j, _):
        di = []
        for b in range(NSLOT):
            base = chunk_base(j * NSLOT + b)
            di.append(pltpu.async_copy(row_hbm.at[pl.ds(base, CHS)],
                                       ridx[b], semi[b]))
            di.append(pltpu.async_copy(col_hbm.at[pl.ds(base, CHS)],
                                       cidx[b], semi[b]))
        dg = []
        for b in range(NSLOT):
            di[2 * b].wait()
            di[2 * b + 1].wait()
            adjust(b)

            @pl.when(j > 0)
            def _():
                drain_scatters(b)

            dg.append(pltpu.async_copy(gflat_hbm.at[radj[b]], bufa[b],
                                       semg[b]))
            dg.append(pltpu.async_copy(gflat_hbm.at[cadj[b]], bufb[b],
                                       semh[b]))
        for b in range(NSLOT):
            def cp(k, _):
                sl = pl.ds(k * 16, 16)
                sridx[b][sl] = ridx[b][sl]
                scidx[b][sl] = cidx[b][sl]
                return 0

            lax.fori_loop(0, CHS // 16, cp, 0)
            dg[2 * b].wait()
            pltpu.async_copy(bufa[b], acc_sh.at[scidx[b]], sems, add=True)
            dg[2 * b + 1].wait()
            pltpu.async_copy(bufb[b], acc_sh.at[sridx[b]], sems, add=True)
        return 0

    lax.fori_loop(0, NBODY_SC, body, 0)
    for b in range(NSLOT):
        drain_scatters(b)

    tbase = s * EPT_SC + NCH_SC * CHS
    pltpu.sync_copy(row_hbm.at[pl.ds(tbase, CHT)], tidx[0])
    pltpu.sync_copy(col_hbm.at[pl.ds(tbase, CHT)], tidx[1])

    def tadj(k, _):
        sl = pl.ds(k * 16, 16)
        tidx[2][sl] = tidx[0][sl] * 2 + c
        tidx[3][sl] = tidx[1][sl] * 2 + c
        return 0

    lax.fori_loop(0, CHT // 16, tadj, 0)
    cpa = pltpu.async_copy(gflat_hbm.at[tidx[2]], tbuf[0], semg[0])
    cpb = pltpu.async_copy(gflat_hbm.at[tidx[3]], tbuf[1], semg[1])
    cpa.wait()
    cpb.wait()
    pltpu.sync_copy(tbuf[0], acc_sh.at[tidx[1]], add=True)
    pltpu.sync_copy(tbuf[1], acc_sh.at[tidx[0]], add=True)

    plsc.subcore_barrier()
    pltpu.sync_copy(acc_sh.at[pl.ds(s * ROWS_PT, ROWS_PT)],
                    out_hbm.at[pl.ds(s * ROWS_PT, ROWS_PT), c])


BR = 2000


def _mm_body(x_ref, wt_ref, b_ref, h_ref):
    h_ref[...] = jnp.dot(x_ref[...], wt_ref[...],
                         preferred_element_type=jnp.float32) + b_ref[...]


def _mm(x, wt, b2):
    return pl.pallas_call(
        _mm_body,
        grid=(N // BR,),
        in_specs=[
            pl.BlockSpec((BR, D), lambda i: (i, 0)),
            pl.BlockSpec((D, D), lambda i: (0, 0)),
            pl.BlockSpec((1, D), lambda i: (0, 0)),
        ],
        out_specs=pl.BlockSpec((BR, D), lambda i: (i, 0)),
        out_shape=jax.ShapeDtypeStruct((N, D), jnp.float32),
    )(x, wt, b2)


def _scale_body(h_ref, degp_ref, g_ref, dis_ref):
    deg = degp_ref[0] + degp_ref[1] + 1.0
    dis = lax.rsqrt(deg)
    dis_ref[...] = dis
    g_ref[...] = h_ref[...] * dis


def _scale(h, degp3):
    return pl.pallas_call(
        _scale_body,
        grid=(N // BR,),
        in_specs=[
            pl.BlockSpec((BR, D), lambda i: (i, 0)),
            pl.BlockSpec((NC, BR, 1), lambda i: (0, i, 0)),
        ],
        out_specs=[
            pl.BlockSpec((BR, D), lambda i: (i, 0)),
            pl.BlockSpec((BR, 1), lambda i: (i, 0)),
        ],
        out_shape=[
            jax.ShapeDtypeStruct((N, D), jnp.float32),
            jax.ShapeDtypeStruct((N, 1), jnp.float32),
        ],
    )(h, degp3)


def _final_body(acc_ref, g_ref, dis_ref, o_ref):
    tot = (acc_ref[...] + g_ref[...]) * dis_ref[...]
    o_ref[...] = jnp.maximum(tot, 0.0)


def _finalize(accv, g, dis):
    return pl.pallas_call(
        _final_body,
        grid=(N // BR,),
        in_specs=[
            pl.BlockSpec((BR, D), lambda i: (i, 0)),
            pl.BlockSpec((BR, D), lambda i: (i, 0)),
            pl.BlockSpec((BR, 1), lambda i: (i, 0)),
        ],
        out_specs=pl.BlockSpec((BR, D), lambda i: (i, 0)),
        out_shape=jax.ShapeDtypeStruct((N, D), jnp.float32),
    )(accv, g, dis)


def kernel(x, edge_index, W, b):
    ei = edge_index.astype(jnp.int32)
    row = ei[0]
    col = ei[1]

    h = _mm(x, W.T, b.reshape(1, D))
    degp = _deg_sc(row, col)

    g, dis = _scale(h, degp.reshape(NC, NPAD, 1))
    gflat = g.reshape(NC * N, HD)
    accp = _scat_sc(row, col, gflat)
    accv = accp.reshape(NPAD, D)
    return _finalize(accv, g, dis)


# speedup vs baseline: 1.0915x; 1.0915x over previous
import functools

import jax
import jax.numpy as jnp
from jax import lax
from jax.experimental import pallas as pl
from jax.experimental.pallas import tpu as pltpu
from jax.experimental.pallas import tpu_sc as plsc

N = 10000
E = 320000
D = 128
HD = D // 2

NC = 2
NS = 16
NW = NC * NS

CH = 80
CHS = 128
NSLOT = 4

NPAD = 10240
DEG_PT = NPAD // NS
ROWS_PT = NPAD // NS
ZROWS = 64

EPT_DEG = E // NW
NCH_DEG = EPT_DEG // CH
NBODY_DEG = NCH_DEG // NSLOT
NTAIL_DEG = NCH_DEG - NBODY_DEG * NSLOT

EPT_SC = E // NS
NCH_SC = EPT_SC // CHS
NBODY_SC = NCH_SC // NSLOT
NTAIL_SC = NCH_SC - NBODY_SC * NSLOT
CHT = EPT_SC - NCH_SC * CHS

_mesh = plsc.VectorSubcoreMesh(core_axis_name="c", subcore_axis_name="s")


@functools.partial(
    pl.kernel,
    out_type=jax.ShapeDtypeStruct((NC, NPAD), jnp.float32),
    mesh=_mesh,
    scratch_types=(
        [pltpu.VMEM((CH,), jnp.int32)] * (2 * NSLOT)
        + [
            pltpu.VMEM((CH,), jnp.float32),
            pltpu.VMEM((DEG_PT,), jnp.float32),
            pltpu.VMEM_SHARED((NPAD,), jnp.float32),
        ]
        + [pltpu.SemaphoreType.DMA] * (NSLOT + 1)
    ),
)
def _deg_sc(row_hbm, col_hbm, out_hbm, *refs):
    ridx = refs[0:NSLOT]
    cidx = refs[NSLOT:2 * NSLOT]
    ones_v, zeros_v, deg_sh = refs[2 * NSLOT:2 * NSLOT + 3]
    semi = refs[2 * NSLOT + 3:3 * NSLOT + 3]
    sems = refs[3 * NSLOT + 3]

    c = lax.axis_index("c")
    s = lax.axis_index("s")
    wid = s * NC + c

    def fill(i, _):
        ones_v[pl.ds(i * 16, 16)] = jnp.ones((16,), jnp.float32)
        return 0

    lax.fori_loop(0, CH // 16, fill, 0)

    def zfill(i, _):
        zeros_v[pl.ds(i * 16, 16)] = jnp.zeros((16,), jnp.float32)
        return 0

    lax.fori_loop(0, DEG_PT // 16, zfill, 0)
    pltpu.sync_copy(zeros_v, deg_sh.at[pl.ds(s * DEG_PT, DEG_PT)])
    plsc.subcore_barrier()

    def chunk_base(i):
        return wid * EPT_DEG + i * CH

    def body(j, _):
        di = []
        for b in range(NSLOT):
            base = chunk_base(j * NSLOT + b)
            di.append(pltpu.async_copy(row_hbm.at[pl.ds(base, CH)],
                                       ridx[b], semi[b]))
            di.append(pltpu.async_copy(col_hbm.at[pl.ds(base, CH)],
                                       cidx[b], semi[b]))
        sc = []
        for b in range(NSLOT):
            di[2 * b].wait()
            di[2 * b + 1].wait()
            sc.append(pltpu.async_copy(ones_v, deg_sh.at[ridx[b]], sems,
                                       add=True))
            sc.append(pltpu.async_copy(ones_v, deg_sh.at[cidx[b]], sems,
                                       add=True))
        for d in sc:
            d.wait()
        return 0

    lax.fori_loop(0, NBODY_DEG, body, 0)

    for t in range(NTAIL_DEG):
        base = chunk_base(NBODY_DEG * NSLOT + t)
        pltpu.sync_copy(row_hbm.at[pl.ds(base, CH)], ridx[0])
        pltpu.sync_copy(col_hbm.at[pl.ds(base, CH)], cidx[0])
        pltpu.sync_copy(ones_v, deg_sh.at[ridx[0]], add=True)
        pltpu.sync_copy(ones_v, deg_sh.at[cidx[0]], add=True)

    plsc.subcore_barrier()
    pltpu.sync_copy(deg_sh.at[pl.ds(s * DEG_PT, DEG_PT)],
                    out_hbm.at[c, pl.ds(s * DEG_PT, DEG_PT)])


@functools.partial(
    pl.kernel,
    out_type=jax.ShapeDtypeStruct((NPAD, D), jnp.float32),
    mesh=_mesh,
    compiler_params=pltpu.CompilerParams(use_tc_tiling_on_sc=False),
    scratch_types=(
        [pltpu.VMEM((CHS,), jnp.int32)] * (6 * NSLOT)
        + [pltpu.VMEM((CHS, HD), jnp.float32)] * (2 * NSLOT)
        + [pltpu.VMEM((CHT,), jnp.int32)] * 4
        + [pltpu.VMEM((CHT, HD), jnp.float32)] * 2
        + [
            pltpu.VMEM((ZROWS, HD), jnp.float32),
            pltpu.VMEM_SHARED((NPAD, HD), jnp.float32),
        ]
        + [pltpu.SemaphoreType.DMA] * (3 * NSLOT + 1)
    ),
)
def _scat_sc(row_hbm, col_hbm, gflat_hbm, out_hbm, *refs):
    ridx = refs[0:NSLOT]
    cidx = refs[NSLOT:2 * NSLOT]
    radj = refs[2 * NSLOT:3 * NSLOT]
    cadj = refs[3 * NSLOT:4 * NSLOT]
    sridx = refs[4 * NSLOT:5 * NSLOT]
    scidx = refs[5 * NSLOT:6 * NSLOT]
    bufa = refs[6 * NSLOT:7 * NSLOT]
    bufb = refs[7 * NSLOT:8 * NSLOT]
    tidx = refs[8 * NSLOT:8 * NSLOT + 4]
    tbuf = refs[8 * NSLOT + 4:8 * NSLOT + 6]
    zbuf, acc_sh = refs[8 * NSLOT + 6:8 * NSLOT + 8]
    semi = refs[8 * NSLOT + 8:9 * NSLOT + 8]
    semg = refs[9 * NSLOT + 8:10 * NSLOT + 8]
    semh = refs[10 * NSLOT + 8:11 * NSLOT + 8]
    sems = refs[11 * NSLOT + 8]

    c = lax.axis_index("c")
    s = lax.axis_index("s")
    coff = c * HD
    goff = c * N

    def zfill(i, _):
        j = i // (HD // 16)
        k = i % (HD // 16)
        zbuf[j, pl.ds(k * 16, 16)] = jnp.zeros((16,), jnp.float32)
        return 0

    lax.fori_loop(0, ZROWS * (HD // 16), zfill, 0)

    def zinit(t, _):
        pltpu.sync_copy(zbuf, acc_sh.at[pl.ds(s * ROWS_PT + t * ZROWS, ZROWS)])
        return 0

    lax.fori_loop(0, ROWS_PT // ZROWS, zinit, 0)
    plsc.subcore_barrier()

    def chunk_base(i):
        return s * EPT_SC + i * CHS

    def adjust(b):
        def adj(k, _):
            sl = pl.ds(k * 16, 16)
            radj[b][sl] = ridx[b][sl] + goff
            cadj[b][sl] = cidx[b][sl] + goff
            return 0

        lax.fori_loop(0, CHS // 16, adj, 0)

    def drain_scatters(b):
        pltpu.make_async_copy(bufa[b], acc_sh.at[scidx[b]], sems).wait()
        pltpu.make_async_copy(bufb[b], acc_sh.at[sridx[b]], sems).wait()

    def body(j, _):
        di = []
        for b in range(NSLOT):
            base = chunk_base(j * NSLOT + b)
            di.append(pltpu.async_copy(row_hbm.at[pl.ds(base, CHS)],
                                       ridx[b], semi[b]))
            di.append(pltpu.async_copy(col_hbm.at[pl.ds(base, CHS)],
                                       cidx[b], semi[b]))
        dg = []
        for b in range(NSLOT):
            di[2 * b].wait()
            di[2 * b + 1].wait()
            adjust(b)

            @pl.when(j > 0)
            def _():
                drain_scatters(b)

            dg.append(pltpu.async_copy(gflat_hbm.at[radj[b]], bufa[b],
                                       semg[b]))
            dg.append(pltpu.async_copy(gflat_hbm.at[cadj[b]], bufb[b],
                                       semh[b]))
        for b in range(NSLOT):
            def cp(k, _):
                sl = pl.ds(k * 16, 16)
                sridx[b][sl] = ridx[b][sl]
                scidx[b][sl] = cidx[b][sl]
                return 0

            lax.fori_loop(0, CHS // 16, cp, 0)
            dg[2 * b].wait()
            pltpu.async_copy(bufa[b], acc_sh.at[scidx[b]], sems, add=True)
            dg[2 * b + 1].wait()
            pltpu.async_copy(bufb[b], acc_sh.at[sridx[b]], sems, add=True)
        return 0

    lax.fori_loop(0, NBODY_SC, body, 0)
    for b in range(NSLOT):
        drain_scatters(b)

    tbase = s * EPT_SC + NCH_SC * CHS
    pltpu.sync_copy(row_hbm.at[pl.ds(tbase, CHT)], tidx[0])
    pltpu.sync_copy(col_hbm.at[pl.ds(tbase, CHT)], tidx[1])

    def tadj(k, _):
        sl = pl.ds(k * 16, 16)
        tidx[2][sl] = tidx[0][sl] + goff
        tidx[3][sl] = tidx[1][sl] + goff
        return 0

    lax.fori_loop(0, CHT // 16, tadj, 0)
    cpa = pltpu.async_copy(gflat_hbm.at[tidx[2]], tbuf[0], semg[0])
    cpb = pltpu.async_copy(gflat_hbm.at[tidx[3]], tbuf[1], semg[1])
    cpa.wait()
    cpb.wait()
    pltpu.sync_copy(tbuf[0], acc_sh.at[tidx[1]], add=True)
    pltpu.sync_copy(tbuf[1], acc_sh.at[tidx[0]], add=True)

    plsc.subcore_barrier()
    pltpu.sync_copy(acc_sh.at[pl.ds(s * ROWS_PT, ROWS_PT)],
                    out_hbm.at[pl.ds(s * ROWS_PT, ROWS_PT), pl.ds(coff, HD)])


BR = 2000


def _mm_body(x_ref, wt_ref, b_ref, h_ref):
    h_ref[...] = jnp.dot(x_ref[...], wt_ref[...],
                         preferred_element_type=jnp.float32) + b_ref[...]


def _mm(x, wt, b2):
    return pl.pallas_call(
        _mm_body,
        grid=(N // BR,),
        in_specs=[
            pl.BlockSpec((BR, D), lambda i: (i, 0)),
            pl.BlockSpec((D, D), lambda i: (0, 0)),
            pl.BlockSpec((1, D), lambda i: (0, 0)),
        ],
        out_specs=pl.BlockSpec((BR, D), lambda i: (i, 0)),
        out_shape=jax.ShapeDtypeStruct((N, D), jnp.float32),
    )(x, wt, b2)


def _scale_body(h_ref, degp_ref, gg_ref, dis_ref):
    deg = degp_ref[0] + degp_ref[1] + 1.0
    dis = lax.rsqrt(deg)
    dis_ref[...] = dis
    g = h_ref[...] * dis
    gg_ref[0] = g[:, :HD]
    gg_ref[1] = g[:, HD:]


def _scale(h, degp3):
    return pl.pallas_call(
        _scale_body,
        grid=(N // BR,),
        in_specs=[
            pl.BlockSpec((BR, D), lambda i: (i, 0)),
            pl.BlockSpec((NC, BR, 1), lambda i: (0, i, 0)),
        ],
        out_specs=[
            pl.BlockSpec((NC, BR, HD), lambda i: (0, i, 0)),
            pl.BlockSpec((BR, 1), lambda i: (i, 0)),
        ],
        out_shape=[
            jax.ShapeDtypeStruct((NC, N, HD), jnp.float32),
            jax.ShapeDtypeStruct((N, 1), jnp.float32),
        ],
    )(h, degp3)


def _final_body(acc_ref, gg_ref, dis_ref, o_ref):
    g = jnp.concatenate([gg_ref[0], gg_ref[1]], axis=1)
    tot = (acc_ref[...] + g) * dis_ref[...]
    o_ref[...] = jnp.maximum(tot, 0.0)


def _finalize(accv, gg, dis):
    return pl.pallas_call(
        _final_body,
        grid=(N // BR,),
        in_specs=[
            pl.BlockSpec((BR, D), lambda i: (i, 0)),
            pl.BlockSpec((NC, BR, HD), lambda i: (0, i, 0)),
            pl.BlockSpec((BR, 1), lambda i: (i, 0)),
        ],
        out_specs=pl.BlockSpec((BR, D), lambda i: (i, 0)),
        out_shape=jax.ShapeDtypeStruct((N, D), jnp.float32),
    )(accv, gg, dis)


def kernel(x, edge_index, W, b):
    ei = edge_index.astype(jnp.int32)
    row = ei[0]
    col = ei[1]

    h = _mm(x, W.T, b.reshape(1, D))
    degp = _deg_sc(row, col)

    gg, dis = _scale(h, degp.reshape(NC, NPAD, 1))
    gflat = gg.reshape(NC * N, HD)
    accv = _scat_sc(row, col, gflat)
    return _finalize(accv, gg, dis)
